# baseline (speedup 1.0000x reference)
import functools

import jax
import jax.numpy as jnp
from jax import lax
from jax.experimental import pallas as pl
from jax.experimental.pallas import tpu as pltpu
from jax.experimental.pallas import tpu_sc as plsc

N_FEAT = 64
N_WORDS = 100000
BATCH_N = 16384
NC, NS = 2, 16
NW = NC * NS
BPW = BATCH_N // NW
CHUNK = 128
NCHUNK = BPW // CHUNK
LANES = 16


@jax.jit
def _sc_pair_dot(Er, Wt, batch_t):
    mesh = plsc.VectorSubcoreMesh(
        core_axis_name="c", subcore_axis_name="s",
        num_cores=NC, num_subcores=NS,
    )

    @functools.partial(
        pl.kernel,
        out_type=jax.ShapeDtypeStruct((BATCH_N,), jnp.float32),
        mesh=mesh,
        scratch_types=[
            pltpu.VMEM((BPW,), jnp.int32),
            pltpu.VMEM((BPW,), jnp.int32),
            pltpu.VMEM((BPW, N_FEAT), jnp.float32),
            pltpu.VMEM((BPW, N_FEAT), jnp.float32),
            pltpu.VMEM((BPW,), jnp.float32),
            pltpu.SemaphoreType.DMA,
            pltpu.SemaphoreType.DMA,
        ],
        compiler_params=pltpu.CompilerParams(
            needs_layout_passes=False, use_tc_tiling_on_sc=False),
    )
    def k(er_hbm, wt_hbm, b_hbm, out_hbm,
          ri_v, ci_v, er_v, wr_v, out_v, sem_e, sem_w):
        wid = lax.axis_index("s") * NC + lax.axis_index("c")
        base = wid * BPW

        pltpu.sync_copy(b_hbm.at[0, pl.ds(base, BPW)], ri_v)
        pltpu.sync_copy(b_hbm.at[1, pl.ds(base, BPW)], ci_v)

        copies = []
        for c in range(NCHUNK):
            src = pl.ds(c * CHUNK, CHUNK)
            dst = pl.ds(c * CHUNK, CHUNK)
            copies.append(pltpu.async_copy(
                er_hbm.at[ri_v.at[src]], er_v.at[dst], sem_e))
            copies.append(pltpu.async_copy(
                wt_hbm.at[ci_v.at[src]], wr_v.at[dst], sem_w))
        for cp in copies:
            cp.wait()

        lane0 = lax.iota(jnp.int32, LANES) == 0

        def body(p, carry):
            acc = er_v[p, pl.ds(0, LANES)] * wr_v[p, pl.ds(0, LANES)]
            for kk in range(1, N_FEAT // LANES):
                acc = acc + er_v[p, pl.ds(kk * LANES, LANES)] * wr_v[p, pl.ds(kk * LANES, LANES)]
            s = jnp.broadcast_to(jnp.sum(acc, axis=0), (LANES,))
            idx = jnp.broadcast_to(p, (LANES,))
            plsc.store_scatter(out_v, [idx], s, mask=lane0)
            return carry

        lax.fori_loop(0, BPW, body, 0, unroll=False)

        pltpu.sync_copy(out_v, out_hbm.at[pl.ds(base, BPW)])

    return k(Er, Wt, batch_t)


def kernel(batch, E, W):
    Er = E[:N_WORDS]
    Wt = W.T
    bt = batch.astype(jnp.int32).T
    return _sc_pair_dot(Er, Wt, bt)
